# Initial kernel scaffold; baseline (speedup 1.0000x reference)
import functools

import jax
import jax.numpy as jnp
from jax import lax
from jax.experimental import pallas as pl
from jax.experimental.pallas import tpu as pltpu
from jax.experimental.pallas import tpu_sc as plsc

B = 16384
N_DENSE = 13
N_SPARSE = 26
VOCAB = 100000
EMB = 32
N_CROSS = 3
OUT_DIM = 64
X_DIM = N_DENSE + N_SPARSE * EMB

_SC_CORES = 2
_SC_SUBCORES = 16
_NW = _SC_CORES * _SC_SUBCORES

_N_ROWS = B * N_SPARSE
_PER_W = _N_ROWS // _NW
_CHUNK = 1024
_N_CHUNKS = _PER_W // _CHUNK


def _sc_gather(table_flat, flat_idx):
    mesh = plsc.VectorSubcoreMesh(core_axis_name="c", subcore_axis_name="s")

    @functools.partial(
        pl.kernel,
        mesh=mesh,
        out_type=jax.ShapeDtypeStruct((_N_ROWS, EMB), jnp.float32),
        scratch_types=[
            pltpu.VMEM((_CHUNK,), jnp.int32),
            pltpu.VMEM((_CHUNK, EMB), jnp.float32),
            pltpu.SemaphoreType.DMA,
        ],
    )
    def gather_k(table_hbm, idx_hbm, out_hbm, idx_v, rows_v, sem):
        wid = lax.axis_index("s") * _SC_CORES + lax.axis_index("c")
        base = wid * _PER_W

        def chunk_body(i, carry):
            off = base + i * _CHUNK
            pltpu.sync_copy(idx_hbm.at[pl.ds(off, _CHUNK)], idx_v)
            pltpu.async_copy(table_hbm.at[idx_v], rows_v, sem).wait()
            pltpu.sync_copy(rows_v, out_hbm.at[pl.ds(off, _CHUNK)])
            return carry

        lax.fori_loop(0, _N_CHUNKS, chunk_body, 0)

    return gather_k(table_flat, flat_idx)


_BT = 1024


def _dcn_block(inp_ref, emb_ref, cwd_ref, cwe_ref, w1d_ref, w1e_ref, b1_ref,
               w2_ref, b2_ref, w3_ref, b3_ref, wod_ref, sc_ref, out_ref):
    dense = inp_ref[:, :N_DENSE]
    e = emb_ref[...]

    p = (jnp.dot(dense, cwd_ref[...], preferred_element_type=jnp.float32)
         + jnp.dot(e, cwe_ref[...], preferred_element_type=jnp.float32))
    wsum = (jnp.sum(cwd_ref[...], axis=0, keepdims=True)
            + jnp.sum(cwe_ref[...], axis=0, keepdims=True))

    b0 = sc_ref[:, 0:1]
    b1c = sc_ref[:, 1:2]
    b2c = sc_ref[:, 2:3]
    bo_s = sc_ref[:, 3:4]

    p0, p1, p2, q = p[:, 0:1], p[:, 1:2], p[:, 2:3], p[:, 3:4]
    s = 1.0 + p0
    t = jnp.broadcast_to(b0, s.shape)
    a1 = s * p1 + t * wsum[:, 1:2]
    s = s + a1
    t = t + b1c
    a2 = s * p2 + t * wsum[:, 2:3]
    s = s + a2
    t = t + b2c
    cross_logit = s * q + t * wsum[:, 3:4]

    h = jnp.maximum(
        jnp.dot(dense, w1d_ref[...], preferred_element_type=jnp.float32)
        + jnp.dot(e, w1e_ref[...], preferred_element_type=jnp.float32)
        + b1_ref[...], 0.0)
    h = jnp.maximum(
        jnp.dot(h, w2_ref[...], preferred_element_type=jnp.float32)
        + b2_ref[...], 0.0)
    h = jnp.maximum(
        jnp.dot(h, w3_ref[...], preferred_element_type=jnp.float32)
        + b3_ref[...], 0.0)
    d = jnp.dot(h, wod_ref[...], preferred_element_type=jnp.float32)

    out_ref[...] = jax.nn.sigmoid(cross_logit + d + bo_s)


def _tc_dcn(inputs, emb, cwd, cwe, w1d, w1e, b1, w2, b2, w3, b3, wod, sc,
            interpret=False):
    grid = (B // _BT,)

    def full(shape):
        return pl.BlockSpec(shape, lambda i: tuple(0 for _ in shape))

    return pl.pallas_call(
        _dcn_block,
        grid=grid,
        in_specs=[
            pl.BlockSpec((_BT, N_DENSE + N_SPARSE), lambda i: (i, 0)),
            pl.BlockSpec((_BT, N_SPARSE * EMB), lambda i: (i, 0)),
            full(cwd.shape),
            full(cwe.shape),
            full(w1d.shape),
            full(w1e.shape),
            full(b1.shape),
            full(w2.shape),
            full(b2.shape),
            full(w3.shape),
            full(b3.shape),
            full(wod.shape),
            full(sc.shape),
        ],
        out_specs=pl.BlockSpec((_BT, 1), lambda i: (i, 0)),
        out_shape=jax.ShapeDtypeStruct((B, 1), jnp.float32),
        interpret=interpret,
    )(inputs, emb, cwd, cwe, w1d, w1e, b1, w2, b2, w3, b3, wod, sc)


def kernel(inputs, embed_tables, cross_w, cross_b, W1, b1, W2, b2, W3, b3,
           Wo, bo):
    idx = inputs[:, N_DENSE:].astype(jnp.int32)
    flat_idx = (idx + jnp.arange(N_SPARSE, dtype=jnp.int32)[None, :]
                * VOCAB).reshape(-1)
    table_flat = embed_tables.reshape(N_SPARSE * VOCAB, EMB)

    emb = _sc_gather(table_flat, flat_idx).reshape(B, N_SPARSE * EMB)

    cw4 = jnp.concatenate(
        [cross_w[0], cross_w[1], cross_w[2], Wo[:X_DIM]], axis=1)
    cwd, cwe = cw4[:N_DENSE], cw4[N_DENSE:]
    w1d, w1e = W1[:N_DENSE], W1[N_DENSE:]
    wod = Wo[X_DIM:]
    sc = jnp.concatenate([cross_b.reshape(-1), bo.reshape(-1)]).reshape(1, 4)
    return _tc_dcn(inputs, emb, cwd, cwe, w1d, w1e, b1.reshape(1, -1),
                   W2, b2.reshape(1, -1), W3, b3.reshape(1, -1), wod, sc)


# speedup vs baseline: 1.2485x; 1.2485x over previous
import functools

import jax
import jax.numpy as jnp
from jax import lax
from jax.experimental import pallas as pl
from jax.experimental.pallas import tpu as pltpu
from jax.experimental.pallas import tpu_sc as plsc

B = 16384
N_DENSE = 13
N_SPARSE = 26
VOCAB = 100000
EMB = 32
N_CROSS = 3
OUT_DIM = 64
X_DIM = N_DENSE + N_SPARSE * EMB

_SC_CORES = 2
_SC_SUBCORES = 16
_NW = _SC_CORES * _SC_SUBCORES

_N_ROWS = B * N_SPARSE
_PER_W = _N_ROWS // _NW
_IDXW = 128
_SLICES = 8
_CHUNK = _SLICES * _IDXW
_N_CHUNKS = _PER_W // _CHUNK


def _sc_gather(table_flat, flat_idx2d):
    mesh = plsc.VectorSubcoreMesh(core_axis_name="c", subcore_axis_name="s")

    @functools.partial(
        pl.kernel,
        mesh=mesh,
        compiler_params=pltpu.CompilerParams(use_tc_tiling_on_sc=False),
        out_type=jax.ShapeDtypeStruct((_N_ROWS, EMB), jnp.float32),
        scratch_types=[
            pltpu.VMEM((_SLICES, _IDXW), jnp.int32),
            pltpu.VMEM((_CHUNK, EMB), jnp.float32),
            pltpu.SemaphoreType.DMA,
        ],
    )
    def gather_k(table_hbm, idx_hbm, out_hbm, idx_v, rows_v, sem):
        wid = lax.axis_index("s") * _SC_CORES + lax.axis_index("c")
        base = wid * _PER_W

        def chunk_body(i, carry):
            off = base + i * _CHUNK
            pltpu.sync_copy(idx_hbm.at[pl.ds(off // _IDXW, _SLICES)], idx_v)
            for j in range(_SLICES):
                pltpu.async_copy(
                    table_hbm.at[idx_v.at[j]],
                    rows_v.at[pl.ds(j * _IDXW, _IDXW)], sem)
            for j in range(_SLICES):
                pltpu.make_async_copy(
                    table_hbm.at[idx_v.at[j]],
                    rows_v.at[pl.ds(j * _IDXW, _IDXW)], sem).wait()
            pltpu.sync_copy(rows_v, out_hbm.at[pl.ds(off, _CHUNK)])
            return carry

        lax.fori_loop(0, _N_CHUNKS, chunk_body, 0)

    return gather_k(table_flat, flat_idx2d)


_BT = 1024


def _dcn_block(inp_ref, emb_ref, cw_ref, b1_ref, w1_ref, w2_ref, b2_ref,
               w3_ref, b3_ref, wo_ref, sc_ref, out_ref):
    x = jnp.concatenate([inp_ref[:, :N_DENSE], emb_ref[...]], axis=1)

    h = jnp.maximum(
        jnp.dot(x, w1_ref[...], preferred_element_type=jnp.float32)
        + b1_ref[...], 0.0)
    h = jnp.maximum(
        jnp.dot(h, w2_ref[...], preferred_element_type=jnp.float32)
        + b2_ref[...], 0.0)
    dnn = jnp.maximum(
        jnp.dot(h, w3_ref[...], preferred_element_type=jnp.float32)
        + b3_ref[...], 0.0)

    xl = x
    for i in range(N_CROSS):
        alpha = jnp.dot(xl, cw_ref[:, i:i + 1],
                        preferred_element_type=jnp.float32)
        xl = (x * alpha + sc_ref[:, i:i + 1]) + xl

    cat = jnp.concatenate([xl, dnn], axis=1)
    logit = jnp.dot(cat, wo_ref[...],
                    preferred_element_type=jnp.float32) + sc_ref[:, 3:4]
    out_ref[...] = jax.nn.sigmoid(logit)


def _tc_dcn(inputs, emb, cw, b1, w1, w2, b2, w3, b3, wo, sc,
            interpret=False):
    grid = (B // _BT,)

    def full(shape):
        return pl.BlockSpec(shape, lambda i: tuple(0 for _ in shape))

    return pl.pallas_call(
        _dcn_block,
        grid=grid,
        in_specs=[
            pl.BlockSpec((_BT, N_DENSE + N_SPARSE), lambda i: (i, 0)),
            pl.BlockSpec((_BT, N_SPARSE * EMB), lambda i: (i, 0)),
            full(cw.shape),
            full(b1.shape),
            full(w1.shape),
            full(w2.shape),
            full(b2.shape),
            full(w3.shape),
            full(b3.shape),
            full(wo.shape),
            full(sc.shape),
        ],
        out_specs=pl.BlockSpec((_BT, 1), lambda i: (i, 0)),
        out_shape=jax.ShapeDtypeStruct((B, 1), jnp.float32),
        interpret=interpret,
    )(inputs, emb, cw, b1, w1, w2, b2, w3, b3, wo, sc)


def kernel(inputs, embed_tables, cross_w, cross_b, W1, b1, W2, b2, W3, b3,
           Wo, bo):
    idx = inputs[:, N_DENSE:].astype(jnp.int32)
    flat_idx = (idx + jnp.arange(N_SPARSE, dtype=jnp.int32)[None, :]
                * VOCAB).reshape(_N_ROWS // _IDXW, _IDXW)
    table_flat = embed_tables.reshape(N_SPARSE * VOCAB, EMB)

    emb = _sc_gather(table_flat, flat_idx).reshape(B, N_SPARSE * EMB)

    cw = jnp.concatenate([cross_w[0], cross_w[1], cross_w[2]], axis=1)
    sc = jnp.concatenate([cross_b.reshape(-1), bo.reshape(-1)]).reshape(1, 4)
    return _tc_dcn(inputs, emb, cw, b1.reshape(1, -1), W1,
                   W2, b2.reshape(1, -1), W3, b3.reshape(1, -1), Wo, sc)
